# baseline (speedup 1.0000x reference)
import functools

import jax
import jax.numpy as jnp
from jax import lax
from jax.experimental import pallas as pl
from jax.experimental.pallas import tpu as pltpu
from jax.experimental.pallas import tpu_sc as plsc

N = 10000
E = 160000
IN = 256
HID = 512
OUT = 128
D1 = IN + OUT

NC = 2
NS = 16
L = 16

HALF = N // NC
RPS = 5120
NPAD = 10240
K = 80
CCH = 125
E2 = NS * CCH * K

_MESH = plsc.VectorSubcoreMesh(core_axis_name="c", subcore_axis_name="s")


_GDN = lax.GatherDimensionNumbers(
    offset_dims=(), collapsed_slice_dims=(0,), start_index_map=(0,))


def _bcast_lane(v, kk):
    idx = jnp.full((L, 1), kk, jnp.int32)
    return lax.gather(v, idx, _GDN, (1,),
                      mode=lax.GatherScatterMode.PROMISE_IN_BOUNDS)


def _zero_rows(rows_ref, d):
    zv = jnp.zeros((L,), jnp.float32)

    @pl.loop(0, K)
    def _(r):
        for j in range(d // L):
            rows_ref[r, pl.ds(j * L, L)] = zv


DH = 128


def _norm_kernel(row3, col3, w3):
    def body(col_hbm, w_hbm, norm_hbm, colb, wb, normb, degc, zbuf, deg):
        sc = lax.axis_index("c")
        t = lax.axis_index("s")

        pltpu.sync_copy(col_hbm.at[t], colb)
        pltpu.sync_copy(w_hbm.at[t], wb)
        zv = jnp.zeros((L,), jnp.float32)
        for j in range(NPAD // NS // L):
            zbuf[pl.ds(j * L, L)] = zv
        pltpu.sync_copy(zbuf, deg.at[pl.ds(t * (NPAD // NS), NPAD // NS)])
        plsc.subcore_barrier()

        @pl.loop(0, CCH)
        def _(c):
            pltpu.sync_copy(wb.at[c], deg.at[colb.at[c]], add=True)

        plsc.subcore_barrier()

        @pl.loop(0, CCH)
        def _(c):
            pltpu.sync_copy(deg.at[colb.at[c]], degc)
            for jj in range(K // L):
                s = pl.ds(jj * L, L)
                dv = degc[s]
                normb[c, s] = jnp.where(dv > 0.0, wb[c, s] / dv, 0.0)

        @pl.when(sc == 0)
        def _():
            pltpu.sync_copy(normb, norm_hbm.at[t])

    f = pl.kernel(
        body,
        out_type=jax.ShapeDtypeStruct((NS, CCH, K), jnp.float32),
        mesh=_MESH,
        scratch_types=[
            pltpu.VMEM((CCH, K), jnp.int32),
            pltpu.VMEM((CCH, K), jnp.float32),
            pltpu.VMEM((CCH, K), jnp.float32),
            pltpu.VMEM((K,), jnp.float32),
            pltpu.VMEM((NPAD // NS,), jnp.float32),
            pltpu.VMEM_SHARED((NPAD,), jnp.float32),
        ],
    )
    return f(col3, w3)


def _propagate_b(v, row3, col3, norm3, init_row):
    d = OUT

    def body(v_hbm, row_hbm, col_hbm, norm_hbm, b3_hbm, out_hbm,
             rowb, colb, normb, lcolb, rows0, rows1, b3v,
             gsem0, gsem1, acc):
        sc = lax.axis_index("c")
        t = lax.axis_index("s")

        pltpu.sync_copy(row_hbm.at[t], rowb)
        pltpu.sync_copy(col_hbm.at[t], colb)
        pltpu.sync_copy(norm_hbm.at[t], normb)
        pltpu.sync_copy(b3_hbm, b3v)

        @pl.loop(0, K)
        def _(r):
            for j in range(d // L):
                s = pl.ds(j * L, L)
                rows0[r, s] = b3v[s]

        for i in range(RPS // NS // K):
            pltpu.sync_copy(rows0, acc.at[pl.ds(t * (RPS // NS) + i * K, K)])

        @pl.loop(0, CCH)
        def _(c):
            for jj in range(K // L):
                s = pl.ds(jj * L, L)
                cv = colb[c, s]
                lc = cv - sc * HALF
                ok = (lc >= 0) & (lc < HALF)
                lcolb[c, s] = jnp.where(ok, lc, HALF + (cv & 63))

        plsc.subcore_barrier()

        bufs = (rows0, rows1)
        sems = (gsem0, gsem1)

        def scale_scatter(c, buf):
            @pl.loop(0, K // L)
            def _(g):
                nvv = normb[c, pl.ds(g * L, L)]
                for kk in range(L):
                    bn = _bcast_lane(nvv, kk)
                    r = g * L + kk
                    for j in range(d // L):
                        s = pl.ds(j * L, L)
                        buf[r, s] = buf[r, s] * bn

            pltpu.sync_copy(buf, acc.at[lcolb.at[c]], add=True)

        pltpu.async_copy(v_hbm.at[rowb.at[0]], rows0, gsem0)

        @pl.loop(0, CCH, step=2)
        def _(c):
            for b in range(2):
                cc = c + b

                @pl.when(cc < CCH)
                def _():
                    pltpu.make_async_copy(
                        v_hbm.at[rowb.at[cc]], bufs[b], sems[b]).wait()

                    @pl.when(cc + 1 < CCH)
                    def _():
                        pltpu.async_copy(v_hbm.at[rowb.at[cc + 1]],
                                         bufs[1 - b], sems[1 - b])

                    scale_scatter(cc, bufs[b])

        plsc.subcore_barrier()

        for i in range(4):
            base = t * (RPS // NS) + i * 80
            pltpu.sync_copy(acc.at[pl.ds(base, 80)],
                            out_hbm.at[sc, pl.ds(base, 80)])

    f = pl.kernel(
        body,
        out_type=jax.ShapeDtypeStruct((NC, RPS, d), jnp.float32),
        mesh=_MESH,
        scratch_types=[
            pltpu.VMEM((CCH, K), jnp.int32),
            pltpu.VMEM((CCH, K), jnp.int32),
            pltpu.VMEM((CCH, K), jnp.float32),
            pltpu.VMEM((CCH, K), jnp.int32),
            pltpu.VMEM((K, d), jnp.float32),
            pltpu.VMEM((K, d), jnp.float32),
            pltpu.VMEM((d,), jnp.float32),
            pltpu.SemaphoreType.DMA,
            pltpu.SemaphoreType.DMA,
            pltpu.VMEM_SHARED((RPS, d), jnp.float32),
        ],
    )
    return f(v, row3, col3, norm3, init_row)


_RB = 512
_NROWS = N + (-N % _RB)


def _dense_body(a_ref, w1t_ref, b1_ref, w2t_ref, b2_ref, attr_ref, w3t_ref,
                o_ref):
    a1 = a_ref[:, :IN]
    a2 = a_ref[:, IN:]
    h1 = jnp.dot(a1, w1t_ref[...], preferred_element_type=jnp.float32)
    h1 = jnp.maximum(h1 + b1_ref[...], 0.0)
    h2 = jnp.dot(a2, w2t_ref[...], preferred_element_type=jnp.float32)
    h2 = jnp.maximum(h2 + b2_ref[...], 0.0)
    attr = attr_ref[...]
    fp = jnp.sum(h1 * attr, axis=1, keepdims=True)
    lp = jnp.sum(h2 * attr, axis=1, keepdims=True)
    fp = jnp.where(fp >= 0.0, fp, 0.01 * fp)
    lp = jnp.where(lp >= 0.0, lp, 0.01 * lp)
    m = jnp.maximum(fp, lp)
    e0 = jnp.exp(fp - m)
    e1 = jnp.exp(lp - m)
    inv = 1.0 / (e0 + e1)
    comb = (e0 * inv) * h1 + (e1 * inv) * h2
    o_ref[...] = jnp.dot(comb, w3t_ref[...], preferred_element_type=jnp.float32)


def _dense(aggx, w1t, b1r, w2t, b2r, attr, w3t):
    nb = _NROWS // _RB
    return pl.pallas_call(
        _dense_body,
        grid=(nb,),
        in_specs=[
            pl.BlockSpec((_RB, D1), lambda i: (i, 0)),
            pl.BlockSpec((IN, HID), lambda i: (0, 0)),
            pl.BlockSpec((1, HID), lambda i: (0, 0)),
            pl.BlockSpec((OUT, HID), lambda i: (0, 0)),
            pl.BlockSpec((1, HID), lambda i: (0, 0)),
            pl.BlockSpec((1, HID), lambda i: (0, 0)),
            pl.BlockSpec((HID, OUT), lambda i: (0, 0)),
        ],
        out_specs=pl.BlockSpec((_RB, OUT), lambda i: (i, 0)),
        out_shape=jax.ShapeDtypeStruct((_NROWS, OUT), jnp.float32),
    )(aggx, w1t, b1r, w2t, b2r, attr, w3t)


def kernel(x, lp_embed, edge_index, edge_weight, labels, pseudo_labels,
           idx_train, W1, b1, W2, b2, W3, b3, att):
    pad = E2 - E
    row3 = jnp.pad(edge_index[0].astype(jnp.int32),
                   (0, pad)).reshape(NS, CCH, K)
    col3 = jnp.pad(edge_index[1].astype(jnp.int32),
                   (0, pad)).reshape(NS, CCH, K)
    w3_ = jnp.pad(edge_weight, (0, pad)).reshape(NS, CCH, K)

    norm3 = _norm_kernel(row3, col3, w3_)
    zrow = jnp.zeros((DH,), jnp.float32)
    agg0 = _propagate_b(x[:, :DH], row3, col3, norm3, zrow)
    agg1 = _propagate_b(x[:, DH:], row3, col3, norm3, zrow)
    agg2 = _propagate_b(lp_embed, row3, col3, norm3, zrow)
    agg_u = jnp.concatenate([agg0, agg1, agg2], axis=2)
    aggx = jnp.concatenate(
        [agg_u[0, :HALF], agg_u[1, :HALF],
         jnp.zeros((_NROWS - N, D1), jnp.float32)], axis=0)

    xw3 = _dense(aggx, W1.T, b1.reshape(1, HID), W2.T, b2.reshape(1, HID),
                 att.reshape(1, HID), W3.T)

    out3 = _propagate_b(xw3, row3, col3, norm3, b3)
    return jnp.concatenate([out3[0, :HALF], out3[1, :HALF]], axis=0)


# speedup vs baseline: 1.0300x; 1.0300x over previous
import functools

import jax
import jax.numpy as jnp
from jax import lax
from jax.experimental import pallas as pl
from jax.experimental.pallas import tpu as pltpu
from jax.experimental.pallas import tpu_sc as plsc

N = 10000
E = 160000
IN = 256
HID = 512
OUT = 128
D1 = IN + OUT

NC = 2
NS = 16
L = 16

HALF = N // NC
RPS = 5120
NPAD = 10240
K = 80
CCH = 125
E2 = NS * CCH * K

_MESH = plsc.VectorSubcoreMesh(core_axis_name="c", subcore_axis_name="s")


_GDN = lax.GatherDimensionNumbers(
    offset_dims=(), collapsed_slice_dims=(0,), start_index_map=(0,))


def _bcast_lane(v, kk):
    idx = jnp.full((L, 1), kk, jnp.int32)
    return lax.gather(v, idx, _GDN, (1,),
                      mode=lax.GatherScatterMode.PROMISE_IN_BOUNDS)


def _zero_rows(rows_ref, d):
    zv = jnp.zeros((L,), jnp.float32)

    @pl.loop(0, K)
    def _(r):
        for j in range(d // L):
            rows_ref[r, pl.ds(j * L, L)] = zv


DH = 128


def _fused_input_kernel(x1, x2, xlp, row3, col3, w3):
    d = DH

    def body(x1_hbm, x2_hbm, xlp_hbm, row_hbm, col_hbm, w_hbm,
             norm_hbm, out_hbm,
             rowb, colb, normb, lcolb, rows0, rows1, degc, zbuf,
             gsem0, gsem1, deg, acc):
        sc = lax.axis_index("c")
        t = lax.axis_index("s")

        pltpu.sync_copy(row_hbm.at[t], rowb)
        pltpu.sync_copy(col_hbm.at[t], colb)
        pltpu.sync_copy(w_hbm.at[t], normb)

        zv = jnp.zeros((L,), jnp.float32)
        for j in range(NPAD // NS // L):
            zbuf[pl.ds(j * L, L)] = zv
        pltpu.sync_copy(zbuf, deg.at[pl.ds(t * (NPAD // NS), NPAD // NS)])
        plsc.subcore_barrier()

        @pl.loop(0, CCH)
        def _(c):
            pltpu.sync_copy(normb.at[c], deg.at[colb.at[c]], add=True)

        plsc.subcore_barrier()

        @pl.loop(0, CCH)
        def _(c):
            pltpu.sync_copy(deg.at[colb.at[c]], degc)
            for jj in range(K // L):
                s = pl.ds(jj * L, L)
                dv = degc[s]
                normb[c, s] = jnp.where(dv > 0.0, normb[c, s] / dv, 0.0)

        @pl.when(sc == 0)
        def _():
            pltpu.sync_copy(normb, norm_hbm.at[t])

        @pl.loop(0, CCH)
        def _(c):
            for jj in range(K // L):
                s = pl.ds(jj * L, L)
                cv = colb[c, s]
                lc = cv - sc * HALF
                ok = (lc >= 0) & (lc < HALF)
                lcolb[c, s] = jnp.where(ok, lc, HALF + (cv & 63))

        bufs = (rows0, rows1)
        sems = (gsem0, gsem1)

        def scale_scatter(c, buf):
            @pl.loop(0, K // L)
            def _(g):
                nvv = normb[c, pl.ds(g * L, L)]
                for kk in range(L):
                    bn = _bcast_lane(nvv, kk)
                    r = g * L + kk
                    for j in range(d // L):
                        s = pl.ds(j * L, L)
                        buf[r, s] = buf[r, s] * bn

            pltpu.sync_copy(buf, acc.at[lcolb.at[c]], add=True)

        def one_pass(v_hbm, p):
            @pl.loop(0, K)
            def _(r):
                for j in range(d // L):
                    rows0[r, pl.ds(j * L, L)] = jnp.zeros((L,), jnp.float32)

            for i in range(RPS // NS // K):
                pltpu.sync_copy(rows0,
                                acc.at[pl.ds(t * (RPS // NS) + i * K, K)])
            plsc.subcore_barrier()

            pltpu.async_copy(v_hbm.at[rowb.at[0]], rows0, gsem0)

            @pl.loop(0, CCH, step=2)
            def _(c):
                for b in range(2):
                    cc = c + b

                    @pl.when(cc < CCH)
                    def _():
                        pltpu.make_async_copy(
                            v_hbm.at[rowb.at[cc]], bufs[b], sems[b]).wait()

                        @pl.when(cc + 1 < CCH)
                        def _():
                            pltpu.async_copy(v_hbm.at[rowb.at[cc + 1]],
                                             bufs[1 - b], sems[1 - b])

                        scale_scatter(cc, bufs[b])

            plsc.subcore_barrier()

            for i in range(4):
                base = t * (RPS // NS) + i * 80
                pltpu.sync_copy(acc.at[pl.ds(base, 80)],
                                out_hbm.at[p, sc, pl.ds(base, 80)])
            plsc.subcore_barrier()

        one_pass(x1_hbm, 0)
        one_pass(x2_hbm, 1)
        one_pass(xlp_hbm, 2)

    f = pl.kernel(
        body,
        out_type=[
            jax.ShapeDtypeStruct((NS, CCH, K), jnp.float32),
            jax.ShapeDtypeStruct((3, NC, RPS, d), jnp.float32),
        ],
        mesh=_MESH,
        scratch_types=[
            pltpu.VMEM((CCH, K), jnp.int32),
            pltpu.VMEM((CCH, K), jnp.int32),
            pltpu.VMEM((CCH, K), jnp.float32),
            pltpu.VMEM((CCH, K), jnp.int32),
            pltpu.VMEM((K, d), jnp.float32),
            pltpu.VMEM((K, d), jnp.float32),
            pltpu.VMEM((K,), jnp.float32),
            pltpu.VMEM((NPAD // NS,), jnp.float32),
            pltpu.SemaphoreType.DMA,
            pltpu.SemaphoreType.DMA,
            pltpu.VMEM_SHARED((NPAD,), jnp.float32),
            pltpu.VMEM_SHARED((RPS, d), jnp.float32),
        ],
    )
    return f(x1, x2, xlp, row3, col3, w3)


def _propagate_b(v, row3, col3, norm3, init_row):
    d = OUT

    def body(v_hbm, row_hbm, col_hbm, norm_hbm, b3_hbm, out_hbm,
             rowb, colb, normb, lcolb, rows0, rows1, b3v,
             gsem0, gsem1, acc):
        sc = lax.axis_index("c")
        t = lax.axis_index("s")

        pltpu.sync_copy(row_hbm.at[t], rowb)
        pltpu.sync_copy(col_hbm.at[t], colb)
        pltpu.sync_copy(norm_hbm.at[t], normb)
        pltpu.sync_copy(b3_hbm, b3v)

        @pl.loop(0, K)
        def _(r):
            for j in range(d // L):
                s = pl.ds(j * L, L)
                rows0[r, s] = b3v[s]

        for i in range(RPS // NS // K):
            pltpu.sync_copy(rows0, acc.at[pl.ds(t * (RPS // NS) + i * K, K)])

        @pl.loop(0, CCH)
        def _(c):
            for jj in range(K // L):
                s = pl.ds(jj * L, L)
                cv = colb[c, s]
                lc = cv - sc * HALF
                ok = (lc >= 0) & (lc < HALF)
                lcolb[c, s] = jnp.where(ok, lc, HALF + (cv & 63))

        plsc.subcore_barrier()

        bufs = (rows0, rows1)
        sems = (gsem0, gsem1)

        def scale_scatter(c, buf):
            @pl.loop(0, K // L)
            def _(g):
                nvv = normb[c, pl.ds(g * L, L)]
                for kk in range(L):
                    bn = _bcast_lane(nvv, kk)
                    r = g * L + kk
                    for j in range(d // L):
                        s = pl.ds(j * L, L)
                        buf[r, s] = buf[r, s] * bn

            pltpu.sync_copy(buf, acc.at[lcolb.at[c]], add=True)

        pltpu.async_copy(v_hbm.at[rowb.at[0]], rows0, gsem0)

        @pl.loop(0, CCH, step=2)
        def _(c):
            for b in range(2):
                cc = c + b

                @pl.when(cc < CCH)
                def _():
                    pltpu.make_async_copy(
                        v_hbm.at[rowb.at[cc]], bufs[b], sems[b]).wait()

                    @pl.when(cc + 1 < CCH)
                    def _():
                        pltpu.async_copy(v_hbm.at[rowb.at[cc + 1]],
                                         bufs[1 - b], sems[1 - b])

                    scale_scatter(cc, bufs[b])

        plsc.subcore_barrier()

        for i in range(4):
            base = t * (RPS // NS) + i * 80
            pltpu.sync_copy(acc.at[pl.ds(base, 80)],
                            out_hbm.at[sc, pl.ds(base, 80)])

    f = pl.kernel(
        body,
        out_type=jax.ShapeDtypeStruct((NC, RPS, d), jnp.float32),
        mesh=_MESH,
        scratch_types=[
            pltpu.VMEM((CCH, K), jnp.int32),
            pltpu.VMEM((CCH, K), jnp.int32),
            pltpu.VMEM((CCH, K), jnp.float32),
            pltpu.VMEM((CCH, K), jnp.int32),
            pltpu.VMEM((K, d), jnp.float32),
            pltpu.VMEM((K, d), jnp.float32),
            pltpu.VMEM((d,), jnp.float32),
            pltpu.SemaphoreType.DMA,
            pltpu.SemaphoreType.DMA,
            pltpu.VMEM_SHARED((RPS, d), jnp.float32),
        ],
    )
    return f(v, row3, col3, norm3, init_row)


_RB = 512
_NROWS = N + (-N % _RB)


def _dense_body(a_ref, w1t_ref, b1_ref, w2t_ref, b2_ref, attr_ref, w3t_ref,
                o_ref):
    a1 = a_ref[:, :IN]
    a2 = a_ref[:, IN:]
    h1 = jnp.dot(a1, w1t_ref[...], preferred_element_type=jnp.float32)
    h1 = jnp.maximum(h1 + b1_ref[...], 0.0)
    h2 = jnp.dot(a2, w2t_ref[...], preferred_element_type=jnp.float32)
    h2 = jnp.maximum(h2 + b2_ref[...], 0.0)
    attr = attr_ref[...]
    fp = jnp.sum(h1 * attr, axis=1, keepdims=True)
    lp = jnp.sum(h2 * attr, axis=1, keepdims=True)
    fp = jnp.where(fp >= 0.0, fp, 0.01 * fp)
    lp = jnp.where(lp >= 0.0, lp, 0.01 * lp)
    m = jnp.maximum(fp, lp)
    e0 = jnp.exp(fp - m)
    e1 = jnp.exp(lp - m)
    inv = 1.0 / (e0 + e1)
    comb = (e0 * inv) * h1 + (e1 * inv) * h2
    o_ref[...] = jnp.dot(comb, w3t_ref[...], preferred_element_type=jnp.float32)


def _dense(aggx, w1t, b1r, w2t, b2r, attr, w3t):
    nb = _NROWS // _RB
    return pl.pallas_call(
        _dense_body,
        grid=(nb,),
        in_specs=[
            pl.BlockSpec((_RB, D1), lambda i: (i, 0)),
            pl.BlockSpec((IN, HID), lambda i: (0, 0)),
            pl.BlockSpec((1, HID), lambda i: (0, 0)),
            pl.BlockSpec((OUT, HID), lambda i: (0, 0)),
            pl.BlockSpec((1, HID), lambda i: (0, 0)),
            pl.BlockSpec((1, HID), lambda i: (0, 0)),
            pl.BlockSpec((HID, OUT), lambda i: (0, 0)),
        ],
        out_specs=pl.BlockSpec((_RB, OUT), lambda i: (i, 0)),
        out_shape=jax.ShapeDtypeStruct((_NROWS, OUT), jnp.float32),
    )(aggx, w1t, b1r, w2t, b2r, attr, w3t)


def kernel(x, lp_embed, edge_index, edge_weight, labels, pseudo_labels,
           idx_train, W1, b1, W2, b2, W3, b3, att):
    pad = E2 - E
    row3 = jnp.pad(edge_index[0].astype(jnp.int32),
                   (0, pad)).reshape(NS, CCH, K)
    col3 = jnp.pad(edge_index[1].astype(jnp.int32),
                   (0, pad)).reshape(NS, CCH, K)
    w3_ = jnp.pad(edge_weight, (0, pad)).reshape(NS, CCH, K)

    norm3, agg = _fused_input_kernel(x[:, :DH], x[:, DH:], lp_embed,
                                     row3, col3, w3_)
    agg_u = jnp.concatenate([agg[0], agg[1], agg[2]], axis=2)
    aggx = jnp.concatenate(
        [agg_u[0, :HALF], agg_u[1, :HALF],
         jnp.zeros((_NROWS - N, D1), jnp.float32)], axis=0)

    xw3 = _dense(aggx, W1.T, b1.reshape(1, HID), W2.T, b2.reshape(1, HID),
                 att.reshape(1, HID), W3.T)

    out3 = _propagate_b(xw3, row3, col3, norm3, b3)
    return jnp.concatenate([out3[0, :HALF], out3[1, :HALF]], axis=0)


# speedup vs baseline: 1.1988x; 1.1639x over previous
import functools

import jax
import jax.numpy as jnp
from jax import lax
from jax.experimental import pallas as pl
from jax.experimental.pallas import tpu as pltpu
from jax.experimental.pallas import tpu_sc as plsc

N = 10000
E = 160000
IN = 256
HID = 512
OUT = 128
D1 = IN + OUT

NC = 2
NS = 16
L = 16

HALF = N // NC
RPS = 5120
NPAD = 10240
K = 80
CCH = 125
E2 = NS * CCH * K

_MESH = plsc.VectorSubcoreMesh(core_axis_name="c", subcore_axis_name="s")


_GDN = lax.GatherDimensionNumbers(
    offset_dims=(), collapsed_slice_dims=(0,), start_index_map=(0,))


def _bcast_lane(v, kk):
    idx = jnp.full((L, 1), kk, jnp.int32)
    return lax.gather(v, idx, _GDN, (1,),
                      mode=lax.GatherScatterMode.PROMISE_IN_BOUNDS)


def _zero_rows(rows_ref, d):
    zv = jnp.zeros((L,), jnp.float32)

    @pl.loop(0, K)
    def _(r):
        for j in range(d // L):
            rows_ref[r, pl.ds(j * L, L)] = zv


DH = 128


def _fused_input_kernel(x1, x2, xlp, row3, col3, w3):
    d = DH

    def body(x1_hbm, x2_hbm, xlp_hbm, row_hbm, col_hbm, w_hbm,
             norm_hbm, out_hbm,
             rowb, colb, normb, rows0, rows1, rows2, degc, zbuf,
             gsem0, gsem1, gsem2, deg, acc):
        sc = lax.axis_index("c")
        t = lax.axis_index("s")

        pltpu.sync_copy(row_hbm.at[t], rowb)
        pltpu.sync_copy(col_hbm.at[t], colb)
        pltpu.sync_copy(w_hbm.at[t], normb)

        zv = jnp.zeros((L,), jnp.float32)
        for j in range(NPAD // NS // L):
            zbuf[pl.ds(j * L, L)] = zv
        pltpu.sync_copy(zbuf, deg.at[pl.ds(t * (NPAD // NS), NPAD // NS)])
        plsc.subcore_barrier()

        @pl.loop(0, CCH)
        def _(c):
            pltpu.sync_copy(normb.at[c], deg.at[colb.at[c]], add=True)

        plsc.subcore_barrier()

        @pl.loop(0, CCH)
        def _(c):
            pltpu.sync_copy(deg.at[colb.at[c]], degc)
            for jj in range(K // L):
                s = pl.ds(jj * L, L)
                dv = degc[s]
                normb[c, s] = jnp.where(dv > 0.0, normb[c, s] / dv, 0.0)

        @pl.when(sc == 0)
        def _():
            pltpu.sync_copy(normb, norm_hbm.at[t])

        @pl.loop(0, CCH)
        def _(c):
            for jj in range(K // L):
                s = pl.ds(jj * L, L)
                cv = colb[c, s]
                lc = cv - sc * HALF
                ok = (lc >= 0) & (lc < HALF)
                colb[c, s] = jnp.where(ok, lc, HALF + (cv & 63))

        bufs = (rows0, rows1, rows2)
        sems = (gsem0, gsem1, gsem2)

        def scale_scatter(c, buf):
            @pl.loop(0, K // L)
            def _(g):
                nvv = normb[c, pl.ds(g * L, L)]
                for kk in range(L):
                    bn = _bcast_lane(nvv, kk)
                    r = g * L + kk
                    for j in range(d // L):
                        s = pl.ds(j * L, L)
                        buf[r, s] = buf[r, s] * bn

            pltpu.sync_copy(buf, acc.at[colb.at[c]], add=True)

        def one_pass(v_hbm, p):
            @pl.loop(0, K)
            def _(r):
                for j in range(d // L):
                    rows0[r, pl.ds(j * L, L)] = jnp.zeros((L,), jnp.float32)

            for i in range(RPS // NS // K):
                pltpu.sync_copy(rows0,
                                acc.at[pl.ds(t * (RPS // NS) + i * K, K)])
            plsc.subcore_barrier()

            pltpu.async_copy(v_hbm.at[rowb.at[0]], rows0, gsem0)
            pltpu.async_copy(v_hbm.at[rowb.at[1]], rows1, gsem1)

            @pl.loop(0, CCH, step=3)
            def _(c):
                for b in range(3):
                    cc = c + b

                    @pl.when(cc < CCH)
                    def _():
                        pltpu.make_async_copy(
                            v_hbm.at[rowb.at[cc]], bufs[b], sems[b]).wait()

                        @pl.when(cc + 2 < CCH)
                        def _():
                            pltpu.async_copy(v_hbm.at[rowb.at[cc + 2]],
                                             bufs[(b + 2) % 3],
                                             sems[(b + 2) % 3])

                        scale_scatter(cc, bufs[b])

            plsc.subcore_barrier()

            for i in range(4):
                base = t * (RPS // NS) + i * 80
                pltpu.sync_copy(acc.at[pl.ds(base, 80)],
                                out_hbm.at[p, sc, pl.ds(base, 80)])
            plsc.subcore_barrier()

        one_pass(x1_hbm, 0)
        one_pass(x2_hbm, 1)
        one_pass(xlp_hbm, 2)

    f = pl.kernel(
        body,
        out_type=[
            jax.ShapeDtypeStruct((NS, CCH, K), jnp.float32),
            jax.ShapeDtypeStruct((3, NC, RPS, d), jnp.float32),
        ],
        mesh=_MESH,
        scratch_types=[
            pltpu.VMEM((CCH, K), jnp.int32),
            pltpu.VMEM((CCH, K), jnp.int32),
            pltpu.VMEM((CCH, K), jnp.float32),
            pltpu.VMEM((K, d), jnp.float32),
            pltpu.VMEM((K, d), jnp.float32),
            pltpu.VMEM((K, d), jnp.float32),
            pltpu.VMEM((K,), jnp.float32),
            pltpu.VMEM((NPAD // NS,), jnp.float32),
            pltpu.SemaphoreType.DMA,
            pltpu.SemaphoreType.DMA,
            pltpu.SemaphoreType.DMA,
            pltpu.VMEM_SHARED((NPAD,), jnp.float32),
            pltpu.VMEM_SHARED((RPS, d), jnp.float32),
        ],
    )
    return f(x1, x2, xlp, row3, col3, w3)


def _propagate_b(v, row3, col3, norm3, init_row):
    d = OUT

    def body(v_hbm, row_hbm, col_hbm, norm_hbm, b3_hbm, out_hbm,
             rowb, colb, normb, lcolb, rows0, rows1, b3v,
             gsem0, gsem1, acc):
        sc = lax.axis_index("c")
        t = lax.axis_index("s")

        pltpu.sync_copy(row_hbm.at[t], rowb)
        pltpu.sync_copy(col_hbm.at[t], colb)
        pltpu.sync_copy(norm_hbm.at[t], normb)
        pltpu.sync_copy(b3_hbm, b3v)

        @pl.loop(0, K)
        def _(r):
            for j in range(d // L):
                s = pl.ds(j * L, L)
                rows0[r, s] = b3v[s]

        for i in range(RPS // NS // K):
            pltpu.sync_copy(rows0, acc.at[pl.ds(t * (RPS // NS) + i * K, K)])

        @pl.loop(0, CCH)
        def _(c):
            for jj in range(K // L):
                s = pl.ds(jj * L, L)
                cv = colb[c, s]
                lc = cv - sc * HALF
                ok = (lc >= 0) & (lc < HALF)
                lcolb[c, s] = jnp.where(ok, lc, HALF + (cv & 63))

        plsc.subcore_barrier()

        bufs = (rows0, rows1)
        sems = (gsem0, gsem1)

        def scale_scatter(c, buf):
            @pl.loop(0, K // L)
            def _(g):
                nvv = normb[c, pl.ds(g * L, L)]
                for kk in range(L):
                    bn = _bcast_lane(nvv, kk)
                    r = g * L + kk
                    for j in range(d // L):
                        s = pl.ds(j * L, L)
                        buf[r, s] = buf[r, s] * bn

            pltpu.sync_copy(buf, acc.at[lcolb.at[c]], add=True)

        pltpu.async_copy(v_hbm.at[rowb.at[0]], rows0, gsem0)

        @pl.loop(0, CCH, step=2)
        def _(c):
            for b in range(2):
                cc = c + b

                @pl.when(cc < CCH)
                def _():
                    pltpu.make_async_copy(
                        v_hbm.at[rowb.at[cc]], bufs[b], sems[b]).wait()

                    @pl.when(cc + 1 < CCH)
                    def _():
                        pltpu.async_copy(v_hbm.at[rowb.at[cc + 1]],
                                         bufs[1 - b], sems[1 - b])

                    scale_scatter(cc, bufs[b])

        plsc.subcore_barrier()

        for i in range(4):
            base = t * (RPS // NS) + i * 80
            pltpu.sync_copy(acc.at[pl.ds(base, 80)],
                            out_hbm.at[sc, pl.ds(base, 80)])

    f = pl.kernel(
        body,
        out_type=jax.ShapeDtypeStruct((NC, RPS, d), jnp.float32),
        mesh=_MESH,
        scratch_types=[
            pltpu.VMEM((CCH, K), jnp.int32),
            pltpu.VMEM((CCH, K), jnp.int32),
            pltpu.VMEM((CCH, K), jnp.float32),
            pltpu.VMEM((CCH, K), jnp.int32),
            pltpu.VMEM((K, d), jnp.float32),
            pltpu.VMEM((K, d), jnp.float32),
            pltpu.VMEM((d,), jnp.float32),
            pltpu.SemaphoreType.DMA,
            pltpu.SemaphoreType.DMA,
            pltpu.VMEM_SHARED((RPS, d), jnp.float32),
        ],
    )
    return f(v, row3, col3, norm3, init_row)


_RB = 512
_NROWS = N + (-N % _RB)


def _dense_body(a_ref, w1t_ref, b1_ref, w2t_ref, b2_ref, attr_ref, w3t_ref,
                o_ref):
    a1 = a_ref[:, :IN]
    a2 = a_ref[:, IN:]
    h1 = jnp.dot(a1, w1t_ref[...], preferred_element_type=jnp.float32)
    h1 = jnp.maximum(h1 + b1_ref[...], 0.0)
    h2 = jnp.dot(a2, w2t_ref[...], preferred_element_type=jnp.float32)
    h2 = jnp.maximum(h2 + b2_ref[...], 0.0)
    attr = attr_ref[...]
    fp = jnp.sum(h1 * attr, axis=1, keepdims=True)
    lp = jnp.sum(h2 * attr, axis=1, keepdims=True)
    fp = jnp.where(fp >= 0.0, fp, 0.01 * fp)
    lp = jnp.where(lp >= 0.0, lp, 0.01 * lp)
    m = jnp.maximum(fp, lp)
    e0 = jnp.exp(fp - m)
    e1 = jnp.exp(lp - m)
    inv = 1.0 / (e0 + e1)
    comb = (e0 * inv) * h1 + (e1 * inv) * h2
    o_ref[...] = jnp.dot(comb, w3t_ref[...], preferred_element_type=jnp.float32)


def _dense(aggx, w1t, b1r, w2t, b2r, attr, w3t):
    nb = _NROWS // _RB
    return pl.pallas_call(
        _dense_body,
        grid=(nb,),
        in_specs=[
            pl.BlockSpec((_RB, D1), lambda i: (i, 0)),
            pl.BlockSpec((IN, HID), lambda i: (0, 0)),
            pl.BlockSpec((1, HID), lambda i: (0, 0)),
            pl.BlockSpec((OUT, HID), lambda i: (0, 0)),
            pl.BlockSpec((1, HID), lambda i: (0, 0)),
            pl.BlockSpec((1, HID), lambda i: (0, 0)),
            pl.BlockSpec((HID, OUT), lambda i: (0, 0)),
        ],
        out_specs=pl.BlockSpec((_RB, OUT), lambda i: (i, 0)),
        out_shape=jax.ShapeDtypeStruct((_NROWS, OUT), jnp.float32),
    )(aggx, w1t, b1r, w2t, b2r, attr, w3t)


def kernel(x, lp_embed, edge_index, edge_weight, labels, pseudo_labels,
           idx_train, W1, b1, W2, b2, W3, b3, att):
    pad = E2 - E
    row3 = jnp.pad(edge_index[0].astype(jnp.int32),
                   (0, pad)).reshape(NS, CCH, K)
    col3 = jnp.pad(edge_index[1].astype(jnp.int32),
                   (0, pad)).reshape(NS, CCH, K)
    w3_ = jnp.pad(edge_weight, (0, pad)).reshape(NS, CCH, K)

    norm3, agg = _fused_input_kernel(x[:, :DH], x[:, DH:], lp_embed,
                                     row3, col3, w3_)
    agg_u = jnp.concatenate([agg[0], agg[1], agg[2]], axis=2)
    aggx = jnp.concatenate(
        [agg_u[0, :HALF], agg_u[1, :HALF],
         jnp.zeros((_NROWS - N, D1), jnp.float32)], axis=0)

    xw3 = _dense(aggx, W1.T, b1.reshape(1, HID), W2.T, b2.reshape(1, HID),
                 att.reshape(1, HID), W3.T)

    out3 = _propagate_b(xw3, row3, col3, norm3, b3)
    return jnp.concatenate([out3[0, :HALF], out3[1, :HALF]], axis=0)


# speedup vs baseline: 1.2682x; 1.0579x over previous
import functools

import jax
import jax.numpy as jnp
from jax import lax
from jax.experimental import pallas as pl
from jax.experimental.pallas import tpu as pltpu
from jax.experimental.pallas import tpu_sc as plsc

N = 10000
E = 160000
IN = 256
HID = 512
OUT = 128
D1 = IN + OUT

NC = 2
NS = 16
L = 16

HALF = N // NC
RPS = 5120
NPAD = 10240
K = 80
CCH = 125
E2 = NS * CCH * K

_MESH = plsc.VectorSubcoreMesh(core_axis_name="c", subcore_axis_name="s")


_GDN = lax.GatherDimensionNumbers(
    offset_dims=(), collapsed_slice_dims=(0,), start_index_map=(0,))


def _bcast_lane(v, kk):
    idx = jnp.full((L, 1), kk, jnp.int32)
    return lax.gather(v, idx, _GDN, (1,),
                      mode=lax.GatherScatterMode.PROMISE_IN_BOUNDS)


def _zero_rows(rows_ref, d):
    zv = jnp.zeros((L,), jnp.float32)

    @pl.loop(0, K)
    def _(r):
        for j in range(d // L):
            rows_ref[r, pl.ds(j * L, L)] = zv


DH = 128


def _fused_input_kernel(x1, x2, xlp, row3, col3, w3):
    d = DH

    def body(x1_hbm, x2_hbm, xlp_hbm, row_hbm, col_hbm, w_hbm,
             norm_hbm, out_hbm,
             rowb, colb, normb, rows0, rows1, rows2, degc, zbuf,
             gsem0, gsem1, gsem2, deg, acc):
        sc = lax.axis_index("c")
        t = lax.axis_index("s")

        pltpu.sync_copy(row_hbm.at[t], rowb)
        pltpu.sync_copy(col_hbm.at[t], colb)
        pltpu.sync_copy(w_hbm.at[t], normb)

        zv = jnp.zeros((L,), jnp.float32)
        for j in range(NPAD // NS // L):
            zbuf[pl.ds(j * L, L)] = zv
        pltpu.sync_copy(zbuf, deg.at[pl.ds(t * (NPAD // NS), NPAD // NS)])
        plsc.subcore_barrier()

        @pl.loop(0, CCH)
        def _(c):
            pltpu.sync_copy(normb.at[c], deg.at[colb.at[c]], add=True)

        plsc.subcore_barrier()

        @pl.loop(0, CCH)
        def _(c):
            pltpu.sync_copy(deg.at[colb.at[c]], degc)
            for jj in range(K // L):
                s = pl.ds(jj * L, L)
                dv = degc[s]
                normb[c, s] = jnp.where(dv > 0.0, normb[c, s] / dv, 0.0)

        @pl.when(sc == 0)
        def _():
            pltpu.sync_copy(normb, norm_hbm.at[t])

        @pl.loop(0, CCH)
        def _(c):
            for jj in range(K // L):
                s = pl.ds(jj * L, L)
                cv = colb[c, s]
                lc = cv - sc * HALF
                ok = (lc >= 0) & (lc < HALF)
                colb[c, s] = jnp.where(ok, lc, HALF + (cv & 63))

        bufs = (rows0, rows1, rows2)
        sems = (gsem0, gsem1, gsem2)

        def scale_scatter(c, buf):
            @pl.loop(0, K // L)
            def _(g):
                nvv = normb[c, pl.ds(g * L, L)]
                for kk in range(L):
                    bn = _bcast_lane(nvv, kk)
                    r = g * L + kk
                    for j in range(d // L):
                        s = pl.ds(j * L, L)
                        buf[r, s] = buf[r, s] * bn

            pltpu.sync_copy(buf, acc.at[colb.at[c]], add=True)

        def one_pass(v_hbm, p):
            @pl.loop(0, K)
            def _(r):
                for j in range(d // L):
                    rows0[r, pl.ds(j * L, L)] = jnp.zeros((L,), jnp.float32)

            for i in range(RPS // NS // K):
                pltpu.sync_copy(rows0,
                                acc.at[pl.ds(t * (RPS // NS) + i * K, K)])
            plsc.subcore_barrier()

            pltpu.async_copy(v_hbm.at[rowb.at[0]], rows0, gsem0)
            pltpu.async_copy(v_hbm.at[rowb.at[1]], rows1, gsem1)

            @pl.loop(0, CCH, step=3)
            def _(c):
                for b in range(3):
                    cc = c + b

                    @pl.when(cc < CCH)
                    def _():
                        pltpu.make_async_copy(
                            v_hbm.at[rowb.at[cc]], bufs[b], sems[b]).wait()

                        @pl.when(cc + 2 < CCH)
                        def _():
                            pltpu.async_copy(v_hbm.at[rowb.at[cc + 2]],
                                             bufs[(b + 2) % 3],
                                             sems[(b + 2) % 3])

                        scale_scatter(cc, bufs[b])

            plsc.subcore_barrier()

            for i in range(4):
                base = t * (RPS // NS) + i * 80
                pltpu.sync_copy(acc.at[pl.ds(base, 80)],
                                out_hbm.at[p, sc, pl.ds(base, 80)])
            plsc.subcore_barrier()

        one_pass(x1_hbm, 0)
        one_pass(x2_hbm, 1)
        one_pass(xlp_hbm, 2)

    f = pl.kernel(
        body,
        out_type=[
            jax.ShapeDtypeStruct((NS, CCH, K), jnp.float32),
            jax.ShapeDtypeStruct((3, NC, RPS, d), jnp.float32),
        ],
        mesh=_MESH,
        scratch_types=[
            pltpu.VMEM((CCH, K), jnp.int32),
            pltpu.VMEM((CCH, K), jnp.int32),
            pltpu.VMEM((CCH, K), jnp.float32),
            pltpu.VMEM((K, d), jnp.float32),
            pltpu.VMEM((K, d), jnp.float32),
            pltpu.VMEM((K, d), jnp.float32),
            pltpu.VMEM((K,), jnp.float32),
            pltpu.VMEM((NPAD // NS,), jnp.float32),
            pltpu.SemaphoreType.DMA,
            pltpu.SemaphoreType.DMA,
            pltpu.SemaphoreType.DMA,
            pltpu.VMEM_SHARED((NPAD,), jnp.float32),
            pltpu.VMEM_SHARED((RPS, d), jnp.float32),
        ],
    )
    return f(x1, x2, xlp, row3, col3, w3)


def _propagate_b(v, row3, col3, norm3, init_row):
    d = OUT

    def body(v_hbm, row_hbm, col_hbm, norm_hbm, b3_hbm, out_hbm,
             rowb, colb, normb, rows0, rows1, rows2, b3v,
             gsem0, gsem1, gsem2, acc):
        sc = lax.axis_index("c")
        t = lax.axis_index("s")

        pltpu.sync_copy(row_hbm.at[t], rowb)
        pltpu.sync_copy(col_hbm.at[t], colb)
        pltpu.sync_copy(norm_hbm.at[t], normb)
        pltpu.sync_copy(b3_hbm, b3v)

        @pl.loop(0, K)
        def _(r):
            for j in range(d // L):
                s = pl.ds(j * L, L)
                rows0[r, s] = b3v[s]

        for i in range(RPS // NS // K):
            pltpu.sync_copy(rows0, acc.at[pl.ds(t * (RPS // NS) + i * K, K)])

        @pl.loop(0, CCH)
        def _(c):
            for jj in range(K // L):
                s = pl.ds(jj * L, L)
                cv = colb[c, s]
                lc = cv - sc * HALF
                ok = (lc >= 0) & (lc < HALF)
                colb[c, s] = jnp.where(ok, lc, HALF + (cv & 63))

        plsc.subcore_barrier()

        bufs = (rows0, rows1, rows2)
        sems = (gsem0, gsem1, gsem2)

        def scale_scatter(c, buf):
            @pl.loop(0, K // L)
            def _(g):
                nvv = normb[c, pl.ds(g * L, L)]
                for kk in range(L):
                    bn = _bcast_lane(nvv, kk)
                    r = g * L + kk
                    for j in range(d // L):
                        s = pl.ds(j * L, L)
                        buf[r, s] = buf[r, s] * bn

            pltpu.sync_copy(buf, acc.at[colb.at[c]], add=True)

        pltpu.async_copy(v_hbm.at[rowb.at[0]], rows0, gsem0)
        pltpu.async_copy(v_hbm.at[rowb.at[1]], rows1, gsem1)

        @pl.loop(0, CCH, step=3)
        def _(c):
            for b in range(3):
                cc = c + b

                @pl.when(cc < CCH)
                def _():
                    pltpu.make_async_copy(
                        v_hbm.at[rowb.at[cc]], bufs[b], sems[b]).wait()

                    @pl.when(cc + 2 < CCH)
                    def _():
                        pltpu.async_copy(v_hbm.at[rowb.at[cc + 2]],
                                         bufs[(b + 2) % 3],
                                         sems[(b + 2) % 3])

                    scale_scatter(cc, bufs[b])

        plsc.subcore_barrier()

        for i in range(4):
            base = t * (RPS // NS) + i * 80
            pltpu.sync_copy(acc.at[pl.ds(base, 80)],
                            out_hbm.at[sc, pl.ds(base, 80)])

    f = pl.kernel(
        body,
        out_type=jax.ShapeDtypeStruct((NC, RPS, d), jnp.float32),
        mesh=_MESH,
        scratch_types=[
            pltpu.VMEM((CCH, K), jnp.int32),
            pltpu.VMEM((CCH, K), jnp.int32),
            pltpu.VMEM((CCH, K), jnp.float32),
            pltpu.VMEM((K, d), jnp.float32),
            pltpu.VMEM((K, d), jnp.float32),
            pltpu.VMEM((K, d), jnp.float32),
            pltpu.VMEM((d,), jnp.float32),
            pltpu.SemaphoreType.DMA,
            pltpu.SemaphoreType.DMA,
            pltpu.SemaphoreType.DMA,
            pltpu.VMEM_SHARED((RPS, d), jnp.float32),
        ],
    )
    return f(v, row3, col3, norm3, init_row)


_RB = 512
_NROWS = N + (-N % _RB)


def _dense_body(a_ref, w1t_ref, b1_ref, w2t_ref, b2_ref, attr_ref, w3t_ref,
                o_ref):
    a1 = a_ref[:, :IN]
    a2 = a_ref[:, IN:]
    h1 = jnp.dot(a1, w1t_ref[...], preferred_element_type=jnp.float32)
    h1 = jnp.maximum(h1 + b1_ref[...], 0.0)
    h2 = jnp.dot(a2, w2t_ref[...], preferred_element_type=jnp.float32)
    h2 = jnp.maximum(h2 + b2_ref[...], 0.0)
    attr = attr_ref[...]
    fp = jnp.sum(h1 * attr, axis=1, keepdims=True)
    lp = jnp.sum(h2 * attr, axis=1, keepdims=True)
    fp = jnp.where(fp >= 0.0, fp, 0.01 * fp)
    lp = jnp.where(lp >= 0.0, lp, 0.01 * lp)
    m = jnp.maximum(fp, lp)
    e0 = jnp.exp(fp - m)
    e1 = jnp.exp(lp - m)
    inv = 1.0 / (e0 + e1)
    comb = (e0 * inv) * h1 + (e1 * inv) * h2
    o_ref[...] = jnp.dot(comb, w3t_ref[...], preferred_element_type=jnp.float32)


def _dense(aggx, w1t, b1r, w2t, b2r, attr, w3t):
    nb = _NROWS // _RB
    return pl.pallas_call(
        _dense_body,
        grid=(nb,),
        in_specs=[
            pl.BlockSpec((_RB, D1), lambda i: (i, 0)),
            pl.BlockSpec((IN, HID), lambda i: (0, 0)),
            pl.BlockSpec((1, HID), lambda i: (0, 0)),
            pl.BlockSpec((OUT, HID), lambda i: (0, 0)),
            pl.BlockSpec((1, HID), lambda i: (0, 0)),
            pl.BlockSpec((1, HID), lambda i: (0, 0)),
            pl.BlockSpec((HID, OUT), lambda i: (0, 0)),
        ],
        out_specs=pl.BlockSpec((_RB, OUT), lambda i: (i, 0)),
        out_shape=jax.ShapeDtypeStruct((_NROWS, OUT), jnp.float32),
    )(aggx, w1t, b1r, w2t, b2r, attr, w3t)


def kernel(x, lp_embed, edge_index, edge_weight, labels, pseudo_labels,
           idx_train, W1, b1, W2, b2, W3, b3, att):
    pad = E2 - E
    row3 = jnp.pad(edge_index[0].astype(jnp.int32),
                   (0, pad)).reshape(NS, CCH, K)
    col3 = jnp.pad(edge_index[1].astype(jnp.int32),
                   (0, pad)).reshape(NS, CCH, K)
    w3_ = jnp.pad(edge_weight, (0, pad)).reshape(NS, CCH, K)

    norm3, agg = _fused_input_kernel(x[:, :DH], x[:, DH:], lp_embed,
                                     row3, col3, w3_)
    agg_u = jnp.concatenate([agg[0], agg[1], agg[2]], axis=2)
    aggx = jnp.concatenate(
        [agg_u[0, :HALF], agg_u[1, :HALF],
         jnp.zeros((_NROWS - N, D1), jnp.float32)], axis=0)

    xw3 = _dense(aggx, W1.T, b1.reshape(1, HID), W2.T, b2.reshape(1, HID),
                 att.reshape(1, HID), W3.T)

    out3 = _propagate_b(xw3, row3, col3, norm3, b3)
    return jnp.concatenate([out3[0, :HALF], out3[1, :HALF]], axis=0)
